# Initial kernel scaffold; baseline (speedup 1.0000x reference)
import jax
import jax.numpy as jnp
from jax.experimental import pallas as pl


def kernel(x, edge_index, batch, W1_0, b1_0, W2_0, b2_0, gamma_0, beta_0, W1_1, b1_1, W2_1, b2_1, gamma_1, beta_1, W1_2, b1_2, W2_2, b2_2, gamma_2, beta_2):
    raise NotImplementedError("write your pallas kernel here")


# speedup vs baseline: 3.9695x; 3.9695x over previous
import functools

import jax
import jax.numpy as jnp
from jax import lax
from jax.experimental import pallas as pl
from jax.experimental.pallas import tpu as pltpu
from jax.experimental.pallas import tpu_sc as plsc

N = 10000
D = 128
G = 64
L = 3
BN_EPS = 1e-5

NC = 2
NS = 16
NW = NC * NS
CHUNK = 128
BLK = 1000
NBLK = N // BLK
N_ACC = 10240
ROWS_PER_TILE = N_ACC // NS
ZCOPY = 128


def _make_agg(chunks_per_worker: int):
    mesh = plsc.VectorSubcoreMesh(core_axis_name="c", subcore_axis_name="s")

    @functools.partial(
        pl.kernel,
        out_type=jax.ShapeDtypeStruct((NC, N_ACC, D), jnp.float32),
        mesh=mesh,
        scratch_types=[
            pltpu.VMEM((chunks_per_worker, CHUNK), jnp.int32),
            pltpu.VMEM((chunks_per_worker, CHUNK), jnp.int32),
            pltpu.VMEM((CHUNK, D), jnp.float32),
            pltpu.VMEM_SHARED((N_ACC, D), jnp.float32),
            pltpu.SemaphoreType.DMA,
        ],
    )
    def agg(ztab_hbm, src_hbm, dst_hbm, out_hbm, src_v, dst_v, rows_v, acc_sh, sem):
        c = lax.axis_index("c")
        s = lax.axis_index("s")
        wid = s * NC + c

        def zrow(r, carry):
            for k in range(D // 16):
                rows_v[r, pl.ds(k * 16, 16)] = jnp.zeros((16,), jnp.float32)
            return carry
        lax.fori_loop(0, CHUNK, zrow, 0)
        base = s * ROWS_PER_TILE
        for k in range(ROWS_PER_TILE // ZCOPY):
            pltpu.sync_copy(rows_v.at[pl.ds(0, ZCOPY)],
                            acc_sh.at[pl.ds(base + k * ZCOPY, ZCOPY)])
        plsc.subcore_barrier()

        pltpu.sync_copy(src_hbm.at[wid], src_v)
        pltpu.sync_copy(dst_hbm.at[wid], dst_v)

        def body(j, carry):
            pltpu.async_copy(ztab_hbm.at[src_v.at[j]], rows_v, sem).wait()
            pltpu.sync_copy(rows_v, acc_sh.at[dst_v.at[j]], add=True)
            return carry
        lax.fori_loop(0, chunks_per_worker, body, 0)
        plsc.subcore_barrier()

        for k in range(ROWS_PER_TILE // ZCOPY):
            off = base + k * ZCOPY
            pltpu.sync_copy(acc_sh.at[pl.ds(off, ZCOPY)], rows_v.at[pl.ds(0, ZCOPY)])
            pltpu.sync_copy(rows_v.at[pl.ds(0, ZCOPY)],
                            out_hbm.at[c, pl.ds(off, ZCOPY)])

    return agg


def _mlp_body(last: bool, z_ref, parts_ref, bt_ref, w1_ref, b1_ref, w2_ref,
              b2_ref, h_ref, g_ref):
    h = z_ref[...] + parts_ref[0] + parts_ref[1]
    h = jnp.maximum(
        jnp.dot(h, w1_ref[...], preferred_element_type=jnp.float32) + b1_ref[...],
        0.0)
    h = jnp.dot(h, w2_ref[...], preferred_element_type=jnp.float32) + b2_ref[...]
    if not last:
        h = jnp.maximum(h, 0.0)
    h_ref[...] = h

    b = bt_ref[0, 0, :]
    oh_t = (lax.broadcasted_iota(jnp.int32, (G, BLK), 0) == b[None, :]
            ).astype(jnp.float32)
    gpart = jnp.dot(oh_t, h, preferred_element_type=jnp.float32)

    @pl.when(pl.program_id(0) == 0)
    def _():
        g_ref[...] = jnp.zeros_like(g_ref)
    g_ref[...] += gpart


def _make_mlp(last: bool):
    return pl.pallas_call(
        functools.partial(_mlp_body, last),
        grid=(NBLK,),
        in_specs=[
            pl.BlockSpec((BLK, D), lambda i: (i, 0)),
            pl.BlockSpec((NC, BLK, D), lambda i: (0, i, 0)),
            pl.BlockSpec((1, 1, BLK), lambda i: (i, 0, 0)),
            pl.BlockSpec((D, D), lambda i: (0, 0)),
            pl.BlockSpec((1, D), lambda i: (0, 0)),
            pl.BlockSpec((D, D), lambda i: (0, 0)),
            pl.BlockSpec((1, D), lambda i: (0, 0)),
        ],
        out_specs=[
            pl.BlockSpec((BLK, D), lambda i: (i, 0)),
            pl.BlockSpec((G, D), lambda i: (0, 0)),
        ],
        out_shape=[
            jax.ShapeDtypeStruct((N, D), jnp.float32),
            jax.ShapeDtypeStruct((G, D), jnp.float32),
        ],
    )


def kernel(x, edge_index, batch,
           W1_0, b1_0, W2_0, b2_0, gamma_0, beta_0,
           W1_1, b1_1, W2_1, b2_1, gamma_1, beta_1,
           W1_2, b1_2, W2_2, b2_2, gamma_2, beta_2):
    params = [
        (W1_0, b1_0, W2_0, b2_0, gamma_0, beta_0),
        (W1_1, b1_1, W2_1, b2_1, gamma_1, beta_1),
        (W1_2, b1_2, W2_2, b2_2, gamma_2, beta_2),
    ]
    src = edge_index[0]
    dst = edge_index[1]
    e = src.shape[0]
    cpw = -(-e // (NW * CHUNK))
    e_pad = NW * cpw * CHUNK
    src_p = jnp.concatenate(
        [src, jnp.full((e_pad - e,), N, jnp.int32)]).reshape(NW, cpw, CHUNK)
    dst_p = jnp.concatenate(
        [dst, jnp.zeros((e_pad - e,), jnp.int32)]).reshape(NW, cpw, CHUNK)
    batch3 = batch.reshape(NBLK, 1, BLK)
    zero_row = jnp.zeros((1, D), jnp.float32)

    agg_fn = _make_agg(cpw)
    mlp_mid = _make_mlp(last=False)
    mlp_last = _make_mlp(last=True)

    z = x
    zs, gs = [], []
    for l in range(L):
        W1, b1, W2, b2, gamma, beta = params[l]
        scale = gamma / jnp.sqrt(1.0 + BN_EPS)
        w2f = W2 * scale[None, :]
        b2f = (b2 * scale + beta).reshape(1, D)
        b1r = b1.reshape(1, D)

        ztab = jnp.concatenate([z, zero_row], axis=0)
        parts = agg_fn(ztab, src_p, dst_p)
        mlp = mlp_last if l == L - 1 else mlp_mid
        h, g = mlp(z, parts, batch3, W1, b1r, w2f, b2f)
        zs.append(h)
        gs.append(g)
        z = h

    return (jnp.concatenate(zs, axis=1), jnp.concatenate(gs, axis=1))
